# baseline (speedup 1.0000x reference)
import functools

import jax
import jax.numpy as jnp
from jax import lax
from jax.experimental import pallas as pl
from jax.experimental.pallas import tpu as pltpu
from jax.experimental.pallas import tpu_sc as plsc

D = 128
BATCH = 2
N = BATCH * D * D * D
NW = 32
PER_W = N // NW
C = 4096
ROWS = C // 128
NCH = PER_W // C
VPC = C // 16


def _warp_body(ddf_hbm, img_hbm, out_hbm, ddf_v, idx_bufs, val_bufs, frac_v, out_v, gsem):
    cid = lax.axis_index("c")
    sid = lax.axis_index("s")
    wid = sid * 2 + cid
    lane = lax.broadcasted_iota(jnp.int32, (16,), 0)

    def chunk(ci, carry):
        vb = wid * PER_W + ci * C

        pltpu.sync_copy(ddf_hbm.at[pl.ds(vb * 3, C * 3)], ddf_v)

        def pass1(i, carry1):
            off = i * 16
            v = vb + off + lane
            b = lax.shift_right_logical(v, 21)
            n = jnp.bitwise_and(v, (1 << 21) - 1)
            x = lax.shift_right_logical(n, 14)
            y = jnp.bitwise_and(lax.shift_right_logical(n, 7), 127)
            z = jnp.bitwise_and(n, 127)
            lo = 3 * off + 3 * lane
            dx = plsc.load_gather(ddf_v, [lo])
            dy = plsc.load_gather(ddf_v, [lo + 1])
            dz = plsc.load_gather(ddf_v, [lo + 2])
            cx = x.astype(jnp.float32) + dx
            cy = y.astype(jnp.float32) + dy
            cz = z.astype(jnp.float32) + dz
            tx = cx.astype(jnp.int32)
            ty = cy.astype(jnp.int32)
            tz = cz.astype(jnp.int32)
            fx = jnp.where(tx.astype(jnp.float32) > cx, tx - 1, tx)
            fy = jnp.where(ty.astype(jnp.float32) > cy, ty - 1, ty)
            fz = jnp.where(tz.astype(jnp.float32) > cz, tz - 1, tz)
            frac_v[0, pl.ds(off, 16)] = cx - fx.astype(jnp.float32)
            frac_v[1, pl.ds(off, 16)] = cy - fy.astype(jnp.float32)
            frac_v[2, pl.ds(off, 16)] = cz - fz.astype(jnp.float32)
            x0 = jnp.clip(fx, 0, D - 1)
            x1 = jnp.clip(fx + 1, 0, D - 1)
            y0 = jnp.clip(fy, 0, D - 1)
            y1 = jnp.clip(fy + 1, 0, D - 1)
            z0 = jnp.clip(fz, 0, D - 1)
            z1 = jnp.clip(fz + 1, 0, D - 1)
            bb = lax.shift_left(b, 21)
            px0 = lax.shift_left(x0, 14)
            px1 = lax.shift_left(x1, 14)
            py0 = lax.shift_left(y0, 7)
            py1 = lax.shift_left(y1, 7)
            a00 = bb + px0 + py0
            a01 = bb + px0 + py1
            a10 = bb + px1 + py0
            a11 = bb + px1 + py1
            idx_bufs[0][pl.ds(off, 16)] = a00 + z0
            idx_bufs[1][pl.ds(off, 16)] = a00 + z1
            idx_bufs[2][pl.ds(off, 16)] = a01 + z0
            idx_bufs[3][pl.ds(off, 16)] = a01 + z1
            idx_bufs[4][pl.ds(off, 16)] = a10 + z0
            idx_bufs[5][pl.ds(off, 16)] = a10 + z1
            idx_bufs[6][pl.ds(off, 16)] = a11 + z0
            idx_bufs[7][pl.ds(off, 16)] = a11 + z1
            return carry1

        lax.fori_loop(0, VPC, pass1, 0)

        copies = [
            pltpu.async_copy(img_hbm.at[idx_bufs[k]], val_bufs[k], gsem)
            for k in range(8)
        ]
        for cp in copies:
            cp.wait()

        def pass2(i, carry2):
            off = i * 16
            fx = frac_v[0, pl.ds(off, 16)]
            fy = frac_v[1, pl.ds(off, 16)]
            fz = frac_v[2, pl.ds(off, 16)]
            gx = 1.0 - fx
            gy = 1.0 - fy
            gz = 1.0 - fz
            w00 = gx * gy
            w01 = gx * fy
            w10 = fx * gy
            w11 = fx * fy
            acc = val_bufs[0][pl.ds(off, 16)] * (w00 * gz)
            acc = acc + val_bufs[1][pl.ds(off, 16)] * (w00 * fz)
            acc = acc + val_bufs[2][pl.ds(off, 16)] * (w01 * gz)
            acc = acc + val_bufs[3][pl.ds(off, 16)] * (w01 * fz)
            acc = acc + val_bufs[4][pl.ds(off, 16)] * (w10 * gz)
            acc = acc + val_bufs[5][pl.ds(off, 16)] * (w10 * fz)
            acc = acc + val_bufs[6][pl.ds(off, 16)] * (w11 * gz)
            acc = acc + val_bufs[7][pl.ds(off, 16)] * (w11 * fz)
            out_v[pl.ds(off, 16)] = acc
            return carry2

        lax.fori_loop(0, VPC, pass2, 0)

        pltpu.sync_copy(out_v, out_hbm.at[pl.ds(vb, C)])
        return carry

    lax.fori_loop(0, NCH, chunk, 0)


_warp = functools.partial(
    pl.kernel,
    out_type=jax.ShapeDtypeStruct((N,), jnp.float32),
    mesh=plsc.VectorSubcoreMesh(core_axis_name="c", subcore_axis_name="s"),
    scratch_types=[
        pltpu.VMEM((3 * C,), jnp.float32),
        [pltpu.VMEM((C,), jnp.int32) for _ in range(8)],
        [pltpu.VMEM((C,), jnp.float32) for _ in range(8)],
        pltpu.VMEM((3, C), jnp.float32),
        pltpu.VMEM((C,), jnp.float32),
        pltpu.SemaphoreType.DMA,
    ],
    compiler_params=pltpu.CompilerParams(needs_layout_passes=False),
)(_warp_body)


def kernel(ddf, image):
    out_flat = _warp(ddf.reshape(-1), image.reshape(-1))
    return out_flat.reshape(BATCH, D, D, D)


# speedup vs baseline: 4.2237x; 4.2237x over previous
import functools

import jax
import jax.numpy as jnp
from jax import lax
from jax.experimental import pallas as pl
from jax.experimental.pallas import tpu as pltpu
from jax.experimental.pallas import tpu_sc as plsc

D = 128
BATCH = 2
N = BATCH * D * D * D
NW = 32
PER_W = N // NW
C = 4096
ROWS = C // 128
NCH = PER_W // C
VPC = C // 16


def _warp_body(ddf0_hbm, ddf1_hbm, ddf2_hbm, img_hbm, out_hbm,
               d0_v, d1_v, d2_v, idx_bufs, val_bufs, frac_v, out_v, gsem):
    cid = lax.axis_index("c")
    sid = lax.axis_index("s")
    wid = sid * 2 + cid
    lane = lax.broadcasted_iota(jnp.int32, (16,), 0)

    def chunk(ci, carry):
        vb = wid * PER_W + ci * C

        pltpu.sync_copy(ddf0_hbm.at[pl.ds(vb, C)], d0_v)
        pltpu.sync_copy(ddf1_hbm.at[pl.ds(vb, C)], d1_v)
        pltpu.sync_copy(ddf2_hbm.at[pl.ds(vb, C)], d2_v)

        def pass1(i, carry1):
            off = i * 16
            v = vb + off + lane
            b = lax.shift_right_logical(v, 21)
            n = jnp.bitwise_and(v, (1 << 21) - 1)
            x = lax.shift_right_logical(n, 14)
            y = jnp.bitwise_and(lax.shift_right_logical(n, 7), 127)
            z = jnp.bitwise_and(n, 127)
            dx = d0_v[pl.ds(off, 16)]
            dy = d1_v[pl.ds(off, 16)]
            dz = d2_v[pl.ds(off, 16)]
            cx = x.astype(jnp.float32) + dx
            cy = y.astype(jnp.float32) + dy
            cz = z.astype(jnp.float32) + dz
            tx = cx.astype(jnp.int32)
            ty = cy.astype(jnp.int32)
            tz = cz.astype(jnp.int32)
            fx = jnp.where(tx.astype(jnp.float32) > cx, tx - 1, tx)
            fy = jnp.where(ty.astype(jnp.float32) > cy, ty - 1, ty)
            fz = jnp.where(tz.astype(jnp.float32) > cz, tz - 1, tz)
            frac_v[0, pl.ds(off, 16)] = cx - fx.astype(jnp.float32)
            frac_v[1, pl.ds(off, 16)] = cy - fy.astype(jnp.float32)
            frac_v[2, pl.ds(off, 16)] = cz - fz.astype(jnp.float32)
            x0 = jnp.clip(fx, 0, D - 1)
            x1 = jnp.clip(fx + 1, 0, D - 1)
            y0 = jnp.clip(fy, 0, D - 1)
            y1 = jnp.clip(fy + 1, 0, D - 1)
            z0 = jnp.clip(fz, 0, D - 1)
            z1 = jnp.clip(fz + 1, 0, D - 1)
            bb = lax.shift_left(b, 21)
            px0 = lax.shift_left(x0, 14)
            px1 = lax.shift_left(x1, 14)
            py0 = lax.shift_left(y0, 7)
            py1 = lax.shift_left(y1, 7)
            a00 = bb + px0 + py0
            a01 = bb + px0 + py1
            a10 = bb + px1 + py0
            a11 = bb + px1 + py1
            idx_bufs[0][pl.ds(off, 16)] = a00 + z0
            idx_bufs[1][pl.ds(off, 16)] = a00 + z1
            idx_bufs[2][pl.ds(off, 16)] = a01 + z0
            idx_bufs[3][pl.ds(off, 16)] = a01 + z1
            idx_bufs[4][pl.ds(off, 16)] = a10 + z0
            idx_bufs[5][pl.ds(off, 16)] = a10 + z1
            idx_bufs[6][pl.ds(off, 16)] = a11 + z0
            idx_bufs[7][pl.ds(off, 16)] = a11 + z1
            return carry1

        lax.fori_loop(0, VPC, pass1, 0)

        copies = [
            pltpu.async_copy(img_hbm.at[idx_bufs[k]], val_bufs[k], gsem)
            for k in range(8)
        ]
        for cp in copies:
            cp.wait()

        def pass2(i, carry2):
            off = i * 16
            fx = frac_v[0, pl.ds(off, 16)]
            fy = frac_v[1, pl.ds(off, 16)]
            fz = frac_v[2, pl.ds(off, 16)]
            gx = 1.0 - fx
            gy = 1.0 - fy
            gz = 1.0 - fz
            w00 = gx * gy
            w01 = gx * fy
            w10 = fx * gy
            w11 = fx * fy
            acc = val_bufs[0][pl.ds(off, 16)] * (w00 * gz)
            acc = acc + val_bufs[1][pl.ds(off, 16)] * (w00 * fz)
            acc = acc + val_bufs[2][pl.ds(off, 16)] * (w01 * gz)
            acc = acc + val_bufs[3][pl.ds(off, 16)] * (w01 * fz)
            acc = acc + val_bufs[4][pl.ds(off, 16)] * (w10 * gz)
            acc = acc + val_bufs[5][pl.ds(off, 16)] * (w10 * fz)
            acc = acc + val_bufs[6][pl.ds(off, 16)] * (w11 * gz)
            acc = acc + val_bufs[7][pl.ds(off, 16)] * (w11 * fz)
            out_v[pl.ds(off, 16)] = acc
            return carry2

        lax.fori_loop(0, VPC, pass2, 0)

        pltpu.sync_copy(out_v, out_hbm.at[pl.ds(vb, C)])
        return carry

    lax.fori_loop(0, NCH, chunk, 0)


_warp = functools.partial(
    pl.kernel,
    out_type=jax.ShapeDtypeStruct((N,), jnp.float32),
    mesh=plsc.VectorSubcoreMesh(core_axis_name="c", subcore_axis_name="s"),
    scratch_types=[
        pltpu.VMEM((C,), jnp.float32),
        pltpu.VMEM((C,), jnp.float32),
        pltpu.VMEM((C,), jnp.float32),
        [pltpu.VMEM((C,), jnp.int32) for _ in range(8)],
        [pltpu.VMEM((C,), jnp.float32) for _ in range(8)],
        pltpu.VMEM((3, C), jnp.float32),
        pltpu.VMEM((C,), jnp.float32),
        pltpu.SemaphoreType.DMA,
    ],
    compiler_params=pltpu.CompilerParams(needs_layout_passes=False),
)(_warp_body)


def kernel(ddf, image):
    d0 = ddf[..., 0].reshape(-1)
    d1 = ddf[..., 1].reshape(-1)
    d2 = ddf[..., 2].reshape(-1)
    out_flat = _warp(d0, d1, d2, image.reshape(-1))
    return out_flat.reshape(BATCH, D, D, D)
